# baseline (speedup 1.0000x reference)
import numpy as np
import jax
import jax.numpy as jnp
from jax import lax
from jax.experimental import pallas as pl
from jax.experimental.pallas import tpu as pltpu

VOCAB = 100000
ROWS, COLS = 128, 4096
TIMESTEPS = 1000

BR, BC = 8, 512
GR, GC = ROWS // BR, COLS // BC

_ROTS = ((13, 15, 26, 6), (17, 29, 16, 24))


def _np_threefry(k0, k1, x0, x1):
    with np.errstate(over="ignore"):
        k0, k1 = np.uint32(k0), np.uint32(k1)
        x0, x1 = np.uint32(x0), np.uint32(x1)
        ks = (k0, k1, np.uint32(k0 ^ k1 ^ np.uint32(0x1BD11BDA)))
        x0 = x0 + ks[0]
        x1 = x1 + ks[1]
        for i in range(5):
            for r in _ROTS[i % 2]:
                x0 = x0 + x1
                x1 = (x1 << np.uint32(r)) | (x1 >> np.uint32(32 - r))
                x1 = x1 ^ x0
            x0 = x0 + ks[(i + 1) % 3]
            x1 = x1 + ks[(i + 2) % 3] + np.uint32(i + 1)
        return x0, x1


def _np_split(k):
    a0, b0 = _np_threefry(k[0], k[1], 0, 0)
    a1, b1 = _np_threefry(k[0], k[1], 0, 1)
    return (a0, b0), (a1, b1)


_KN, _KB = _np_split((np.uint32(0), np.uint32(42)))
_K1, _K2 = _np_split(_KN)


def _u32(v):
    return np.uint32(v & 0xFFFFFFFF)


def _tf_bits(k0, k1, x1_in):
    ks = (_u32(int(k0)), _u32(int(k1)), _u32(int(k0) ^ int(k1) ^ 0x1BD11BDA))
    x1 = x1_in + ks[1]
    x0 = x1_in + _u32(int(ks[0]) + int(ks[1]))
    for i in range(5):
        for j, r in enumerate(_ROTS[i % 2]):
            if i or j:
                x0 = x0 + x1
            x1 = ((x1 << _u32(r)) | (x1 >> _u32(32 - r))) ^ x0
        x0 = x0 + ks[(i + 1) % 3]
        x1 = x1 + _u32(int(ks[(i + 2) % 3]) + i + 1)
    return x0 ^ x1


def _umod_vocab(bits_u32):
    f = bits_u32.astype(jnp.float32)
    q = (f * np.float32((1.0 + 1e-6) / VOCAB)).astype(jnp.uint32)
    r = bits_u32 - q * _u32(VOCAB)
    return jnp.where(r >= _u32(0x80000000), r + _u32(VOCAB), r)


def _thr_body(t_ref, ab_ref, thr_ref):
    t = t_ref[:]
    steps = lax.broadcasted_iota(jnp.int32, (ROWS, TIMESTEPS), 1)
    ab = ab_ref[:]
    a_row = jnp.sum(jnp.where(t == steps, ab, 0.0), axis=1, keepdims=True)
    thr_ref[:] = jnp.ceil(a_row * np.float32(1 << 23)).astype(jnp.uint32)


def _main_body(thr_ref, x_ref, o_ref):
    bi = pl.program_id(0)
    bj = pl.program_id(1)
    row = lax.broadcasted_iota(jnp.int32, (BR, BC), 0) + bi * BR
    col = lax.broadcasted_iota(jnp.int32, (BR, BC), 1) + bj * BC
    idx = (row * COLS + col).astype(jnp.uint32)

    noise = _umod_vocab(_tf_bits(_K2[0], _K2[1], idx)).astype(jnp.int32)
    ubits = _tf_bits(_KB[0], _KB[1], idx)
    keep = (ubits >> _u32(9)) < thr_ref[:]
    o_ref[:] = jnp.where(keep, x_ref[:], noise)


@jax.jit
def kernel(x_start, t, alpha_bars):
    x_start = x_start.astype(jnp.int32)
    t2 = t.astype(jnp.int32).reshape(ROWS, 1)
    ab2 = alpha_bars.astype(jnp.float32).reshape(1, TIMESTEPS)
    thr = pl.pallas_call(
        _thr_body,
        out_shape=jax.ShapeDtypeStruct((ROWS, 1), jnp.uint32),
    )(t2, ab2)
    return pl.pallas_call(
        _main_body,
        grid=(GR, GC),
        in_specs=[
            pl.BlockSpec((BR, 1), lambda i, j: (i, 0)),
            pl.BlockSpec((BR, BC), lambda i, j: (i, j)),
        ],
        out_specs=pl.BlockSpec((BR, BC), lambda i, j: (i, j)),
        out_shape=jax.ShapeDtypeStruct((ROWS, COLS), jnp.int32),
    )(thr, x_start)


# speedup vs baseline: 3.2480x; 3.2480x over previous
import numpy as np
import jax
import jax.numpy as jnp
from jax import lax
from jax.experimental import pallas as pl
from jax.experimental.pallas import tpu as pltpu

VOCAB = 100000
ROWS, COLS = 128, 4096
TIMESTEPS = 1000

BR, BC = 8, 512
GR, GC = ROWS // BR, COLS // BC

_ROTS = ((13, 15, 26, 6), (17, 29, 16, 24))


def _np_threefry(k0, k1, x0, x1):
    with np.errstate(over="ignore"):
        k0, k1 = np.uint32(k0), np.uint32(k1)
        x0, x1 = np.uint32(x0), np.uint32(x1)
        ks = (k0, k1, np.uint32(k0 ^ k1 ^ np.uint32(0x1BD11BDA)))
        x0 = x0 + ks[0]
        x1 = x1 + ks[1]
        for i in range(5):
            for r in _ROTS[i % 2]:
                x0 = x0 + x1
                x1 = (x1 << np.uint32(r)) | (x1 >> np.uint32(32 - r))
                x1 = x1 ^ x0
            x0 = x0 + ks[(i + 1) % 3]
            x1 = x1 + ks[(i + 2) % 3] + np.uint32(i + 1)
        return x0, x1


def _np_split(k):
    a0, b0 = _np_threefry(k[0], k[1], 0, 0)
    a1, b1 = _np_threefry(k[0], k[1], 0, 1)
    return (a0, b0), (a1, b1)


_KN, _KB = _np_split((np.uint32(0), np.uint32(42)))
_K1, _K2 = _np_split(_KN)


def _u32(v):
    return np.uint32(v & 0xFFFFFFFF)


def _tf_bits(k0, k1, x1_in):
    ks = (_u32(int(k0)), _u32(int(k1)), _u32(int(k0) ^ int(k1) ^ 0x1BD11BDA))
    x1 = x1_in + ks[1]
    x0 = x1_in + _u32(int(ks[0]) + int(ks[1]))
    for i in range(5):
        for j, r in enumerate(_ROTS[i % 2]):
            if i or j:
                x0 = x0 + x1
            x1 = ((x1 << _u32(r)) | (x1 >> _u32(32 - r))) ^ x0
        x0 = x0 + ks[(i + 1) % 3]
        x1 = x1 + _u32(int(ks[(i + 2) % 3]) + i + 1)
    return x0 ^ x1


def _umod_vocab(bits_u32):
    f = bits_u32.astype(jnp.float32)
    q = (f * np.float32((1.0 + 1e-6) / VOCAB)).astype(jnp.uint32)
    r = bits_u32 - q * _u32(VOCAB)
    return jnp.where(r >= _u32(0x80000000), r + _u32(VOCAB), r)


def _body(t_ref, ab_ref, x_ref, o_ref):
    t = t_ref[:]
    steps = lax.broadcasted_iota(jnp.int32, (ROWS, TIMESTEPS), 1)
    ab = ab_ref[:]
    a_row = jnp.sum(jnp.where(t == steps, ab, 0.0), axis=1, keepdims=True)
    thr = jnp.ceil(a_row * np.float32(1 << 23)).astype(jnp.uint32)

    iota_r = lax.broadcasted_iota(jnp.int32, (BR, BC), 0)
    iota_c = lax.broadcasted_iota(jnp.int32, (BR, BC), 1)
    for r0 in range(0, ROWS, BR):
        thr_blk = thr[r0:r0 + BR, :]
        for c0 in range(0, COLS, BC):
            idx = ((iota_r + r0) * COLS + (iota_c + c0)).astype(jnp.uint32)
            noise = _umod_vocab(_tf_bits(_K2[0], _K2[1], idx)).astype(jnp.int32)
            ubits = _tf_bits(_KB[0], _KB[1], idx)
            keep = (ubits >> _u32(9)) < thr_blk
            o_ref[r0:r0 + BR, c0:c0 + BC] = jnp.where(
                keep, x_ref[r0:r0 + BR, c0:c0 + BC], noise)


@jax.jit
def kernel(x_start, t, alpha_bars):
    x_start = x_start.astype(jnp.int32)
    t2 = t.astype(jnp.int32).reshape(ROWS, 1)
    ab2 = alpha_bars.astype(jnp.float32).reshape(1, TIMESTEPS)
    return pl.pallas_call(
        _body,
        out_shape=jax.ShapeDtypeStruct((ROWS, COLS), jnp.int32),
    )(t2, ab2, x_start)
